# baseline (speedup 1.0000x reference)
import functools

import jax
import jax.numpy as jnp
from jax import lax
from jax.experimental import pallas as pl
from jax.experimental.pallas import tpu as pltpu
from jax.experimental.pallas import tpu_sc as plsc

NUM_REL = 2212
H = 16
N = 577
NC = 2
NS = 16
L = 16

ROWS_W = 289
CHUNKS = [36] * 7 + [37]
RCMAX = 37
VPR = 37


def _sc_bias_gather(table, idx):
    mesh = plsc.VectorSubcoreMesh(core_axis_name="c", subcore_axis_name="s")

    @functools.partial(
        pl.kernel,
        mesh=mesh,
        out_type=jax.ShapeDtypeStruct((H, N, N), jnp.float32),
        compiler_params=pltpu.CompilerParams(
            needs_layout_passes=False, use_tc_tiling_on_sc=False
        ),
        scratch_types=[
            pltpu.VMEM((NUM_REL * H,), jnp.float32),
            pltpu.VMEM((RCMAX, N), jnp.int32),
            pltpu.VMEM((RCMAX, N), jnp.int32),
            pltpu.VMEM((RCMAX, N), jnp.float32),
            pltpu.VMEM((RCMAX, N), jnp.float32),
            pltpu.SemaphoreType.DMA,
            pltpu.SemaphoreType.DMA,
            pltpu.SemaphoreType.DMA,
            pltpu.SemaphoreType.DMA,
        ],
    )
    def k(table_hbm, idx_hbm, out_hbm, table_v,
          idx0, idx1, buf0, buf1, isem0, isem1, osem0, osem1):
        cid = lax.axis_index("c")
        sid = lax.axis_index("s")
        wid = sid * NC + cid
        h = wid // 2
        half = wid % 2
        row_base = half * (ROWS_W - 1)
        hoff = jnp.full((L,), h * NUM_REL, dtype=jnp.int32)

        pltpu.sync_copy(table_hbm, table_v)

        idxs = (idx0, idx1)
        bufs = (buf0, buf1)
        isems = (isem0, isem1)
        osems = (osem0, osem1)
        out_pending = [None, None]
        idx_pending = [None, None]

        starts = [0]
        for nr in CHUNKS:
            starts.append(starts[-1] + nr)

        idx_pending[0] = pltpu.async_copy(
            idx_hbm.at[pl.ds(row_base, CHUNKS[0]), :],
            idx0.at[pl.ds(0, CHUNKS[0]), :], isem0,
        )

        for c, nr in enumerate(CHUNKS):
            slot = c % 2
            nslot = (c + 1) % 2
            if c + 1 < len(CHUNKS):
                nnr = CHUNKS[c + 1]
                idx_pending[nslot] = pltpu.async_copy(
                    idx_hbm.at[pl.ds(row_base + starts[c + 1], nnr), :],
                    idxs[nslot].at[pl.ds(0, nnr), :], isems[nslot],
                )
            idx_pending[slot].wait()
            if out_pending[slot] is not None:
                out_pending[slot].wait()
            buf = bufs[slot]
            idx_v = idxs[slot]

            @plsc.parallel_loop(0, nr, 1, unroll=1)
            def _(r, buf=buf, idx_v=idx_v):
                for v in range(VPR):
                    off = min(v * L, N - L)
                    iv = idx_v[r, pl.ds(off, L)]
                    vals = plsc.load_gather(table_v, [iv + hoff])
                    buf[r, pl.ds(off, L)] = vals

            out_pending[slot] = pltpu.async_copy(
                buf.at[pl.ds(0, nr), :],
                out_hbm.at[h, pl.ds(row_base + starts[c], nr), :],
                osems[slot],
            )

        for p in out_pending:
            if p is not None:
                p.wait()

    return k(table, idx)


def kernel(relative_position_bias_table, relative_position_index):
    table = relative_position_bias_table.astype(jnp.float32).T.reshape(-1)
    idx = relative_position_index.astype(jnp.int32)
    return _sc_bias_gather(table, idx)


# speedup vs baseline: 1.1122x; 1.1122x over previous
import functools

import jax
import jax.numpy as jnp
from jax import lax
from jax.experimental import pallas as pl
from jax.experimental.pallas import tpu as pltpu
from jax.experimental.pallas import tpu_sc as plsc

NUM_REL = 2212
H = 16
N = 577
NC = 2
NS = 16
L = 16

ROWS_W = 19
RSTRIDE = 18
CHUNKS = [4, 4, 4, 4, 3]
RCMAX = 4
VPR = 37


def _sc_bias_gather(table, idx):
    mesh = plsc.VectorSubcoreMesh(core_axis_name="c", subcore_axis_name="s")

    @functools.partial(
        pl.kernel,
        mesh=mesh,
        out_type=jax.ShapeDtypeStruct((H, N, N), jnp.float32),
        compiler_params=pltpu.CompilerParams(
            needs_layout_passes=False, use_tc_tiling_on_sc=False
        ),
        scratch_types=[
            pltpu.VMEM((NUM_REL * H,), jnp.float32),
            pltpu.VMEM((RCMAX, N), jnp.int32),
            pltpu.VMEM((RCMAX, N), jnp.int32),
            pltpu.VMEM((H, RCMAX, N), jnp.float32),
            pltpu.VMEM((H, RCMAX, N), jnp.float32),
            pltpu.SemaphoreType.DMA,
            pltpu.SemaphoreType.DMA,
            pltpu.SemaphoreType.DMA,
            pltpu.SemaphoreType.DMA,
        ],
    )
    def k(table_hbm, idx_hbm, out_hbm, table_v,
          idx0, idx1, buf0, buf1, isem0, isem1, osem0, osem1):
        cid = lax.axis_index("c")
        sid = lax.axis_index("s")
        wid = sid * NC + cid
        row_base = jnp.minimum(wid * RSTRIDE, N - ROWS_W)

        pltpu.sync_copy(table_hbm, table_v)

        idxs = (idx0, idx1)
        bufs = (buf0, buf1)
        isems = (isem0, isem1)
        osems = (osem0, osem1)
        out_pending = [None, None]
        idx_pending = [None, None]

        starts = [0]
        for nr in CHUNKS:
            starts.append(starts[-1] + nr)

        idx_pending[0] = pltpu.async_copy(
            idx_hbm.at[pl.ds(row_base, CHUNKS[0]), :],
            idx0.at[pl.ds(0, CHUNKS[0]), :], isem0,
        )

        for c, nr in enumerate(CHUNKS):
            slot = c % 2
            nslot = (c + 1) % 2
            if c + 1 < len(CHUNKS):
                nnr = CHUNKS[c + 1]
                idx_pending[nslot] = pltpu.async_copy(
                    idx_hbm.at[pl.ds(row_base + starts[c + 1], nnr), :],
                    idxs[nslot].at[pl.ds(0, nnr), :], isems[nslot],
                )
            idx_pending[slot].wait()
            if out_pending[slot] is not None:
                for p in out_pending[slot]:
                    p.wait()
            buf = bufs[slot]
            idx_v = idxs[slot]

            @plsc.parallel_loop(0, VPR, 1, unroll=1)
            def _(v, buf=buf, idx_v=idx_v, nr=nr):
                off = jnp.minimum(v * L, N - L)
                for r in range(nr):
                    iv = idx_v[r, pl.ds(off, L)]
                    for h in range(H):
                        vals = plsc.load_gather(table_v, [iv + h * NUM_REL])
                        buf[h, r, pl.ds(off, L)] = vals

            cps = []
            for h in range(H):
                cps.append(pltpu.async_copy(
                    buf.at[h, pl.ds(0, nr), :],
                    out_hbm.at[h, pl.ds(row_base + starts[c], nr), :],
                    osems[slot],
                ))
            out_pending[slot] = cps

        for ps in out_pending:
            if ps is not None:
                for p in ps:
                    p.wait()

    return k(table, idx)


def kernel(relative_position_bias_table, relative_position_index):
    table = relative_position_bias_table.astype(jnp.float32).T.reshape(-1)
    idx = relative_position_index.astype(jnp.int32)
    return _sc_bias_gather(table, idx)


# speedup vs baseline: 2.1383x; 1.9226x over previous
import functools

import jax
import jax.numpy as jnp
from jax import lax
from jax.experimental import pallas as pl
from jax.experimental.pallas import tpu as pltpu
from jax.experimental.pallas import tpu_sc as plsc

NUM_REL = 2212
H = 16
N = 577
NPAD = 584
NC = 2
NS = 16
L = 16

RG = 8
HG = 8
NRG = 3
VPR = 37


def _sc_bias_gather(table, idx):
    mesh = plsc.VectorSubcoreMesh(core_axis_name="c", subcore_axis_name="s")

    @functools.partial(
        pl.kernel,
        mesh=mesh,
        out_type=jax.ShapeDtypeStruct((H, NPAD, N), jnp.float32),
        compiler_params=pltpu.CompilerParams(needs_layout_passes=False),
        scratch_types=[
            pltpu.VMEM((NUM_REL * H,), jnp.float32),
            pltpu.VMEM((RG, N), jnp.int32),
            pltpu.VMEM((RG, N), jnp.int32),
            pltpu.VMEM((HG, RG, N), jnp.float32),
            pltpu.VMEM((HG, RG, N), jnp.float32),
            pltpu.SemaphoreType.DMA,
            pltpu.SemaphoreType.DMA,
            pltpu.SemaphoreType.DMA,
            pltpu.SemaphoreType.DMA,
        ],
    )
    def k(table_hbm, idx_hbm, out_hbm, table_v,
          idx0, idx1, buf0, buf1, isem0, isem1, osem0, osem1):
        cid = lax.axis_index("c")
        sid = lax.axis_index("s")
        wid = sid * NC + cid
        row_base = ((wid * 70) // 31) * 8

        pltpu.sync_copy(table_hbm, table_v)

        idxs = (idx0, idx1)
        bufs = (buf0, buf1)
        isems = (isem0, isem1)
        osems = (osem0, osem1)
        out_pending = [None, None]
        idx_pending = [None, None]

        idx_pending[0] = pltpu.async_copy(
            idx_hbm.at[pl.ds(row_base, RG), :], idx0, isem0,
        )

        step = 0
        for rg in range(NRG):
            rslot = rg % 2
            if rg + 1 < NRG:
                idx_pending[(rg + 1) % 2] = pltpu.async_copy(
                    idx_hbm.at[pl.ds(row_base + (rg + 1) * RG, RG), :],
                    idxs[(rg + 1) % 2], isems[(rg + 1) % 2],
                )
            idx_pending[rslot].wait()
            idx_v = idxs[rslot]
            for hg in range(2):
                slot = step % 2
                step += 1
                if out_pending[slot] is not None:
                    out_pending[slot].wait()
                buf = bufs[slot]
                hbase = hg * HG * NUM_REL

                @plsc.parallel_loop(0, VPR, 1, unroll=1)
                def _(v, buf=buf, idx_v=idx_v, hbase=hbase):
                    off = jnp.minimum(v * L, N - L)
                    for r in range(RG):
                        iv = idx_v[r, pl.ds(off, L)] + hbase
                        for h in range(HG):
                            vals = plsc.load_gather(
                                table_v, [iv + h * NUM_REL])
                            buf[h, r, pl.ds(off, L)] = vals

                out_pending[slot] = pltpu.async_copy(
                    buf,
                    out_hbm.at[pl.ds(hg * HG, HG),
                               pl.ds(row_base + rg * RG, RG), :],
                    osems[slot],
                )

        for p in out_pending:
            if p is not None:
                p.wait()

    return k(table, idx)


def kernel(relative_position_bias_table, relative_position_index):
    table = relative_position_bias_table.astype(jnp.float32).T.reshape(-1)
    idx = relative_position_index.astype(jnp.int32)
    idx = jnp.pad(idx, ((0, NPAD - N), (0, 0)))
    out = _sc_bias_gather(table, idx)
    return out[:, :N, :]


# speedup vs baseline: 2.2665x; 1.0599x over previous
import functools

import jax
import jax.numpy as jnp
from jax import lax
from jax.experimental import pallas as pl
from jax.experimental.pallas import tpu as pltpu
from jax.experimental.pallas import tpu_sc as plsc

NUM_REL = 2212
H = 16
N = 577
NPAD = 584
NC = 2
NS = 16
L = 16

RG = 8
HG = 8
NRG = 3
VPR = 37


def _sc_bias_gather(table, idx):
    mesh = plsc.VectorSubcoreMesh(core_axis_name="c", subcore_axis_name="s")

    @functools.partial(
        pl.kernel,
        mesh=mesh,
        out_type=jax.ShapeDtypeStruct((H, N, N), jnp.float32),
        compiler_params=pltpu.CompilerParams(needs_layout_passes=False),
        scratch_types=[
            pltpu.VMEM((NUM_REL * H,), jnp.float32),
            pltpu.VMEM((RG, N), jnp.int32),
            pltpu.VMEM((RG, N), jnp.int32),
            pltpu.VMEM((HG, RG, N), jnp.float32),
            pltpu.VMEM((HG, RG, N), jnp.float32),
            pltpu.SemaphoreType.DMA,
            pltpu.SemaphoreType.DMA,
            pltpu.SemaphoreType.DMA,
            pltpu.SemaphoreType.DMA,
        ],
    )
    def k(table_hbm, idx_hbm, out_hbm, table_v,
          idx0, idx1, buf0, buf1, isem0, isem1, osem0, osem1):
        cid = lax.axis_index("c")
        sid = lax.axis_index("s")
        wid = sid * NC + cid
        row_base = ((wid * 70) // 31) * 8

        pltpu.sync_copy(table_hbm, table_v)

        idxs = (idx0, idx1)
        bufs = (buf0, buf1)
        isems = (isem0, isem1)
        osems = (osem0, osem1)
        out_pending = [None, None]
        idx_pending = [None, None]

        idx_pending[0] = pltpu.async_copy(
            idx_hbm.at[pl.ds(row_base, RG), :], idx0, isem0,
        )

        step = 0
        for rg in range(NRG):
            rslot = rg % 2
            if rg + 1 < NRG:
                idx_pending[(rg + 1) % 2] = pltpu.async_copy(
                    idx_hbm.at[pl.ds(row_base + (rg + 1) * RG, RG), :],
                    idxs[(rg + 1) % 2], isems[(rg + 1) % 2],
                )
            idx_pending[rslot].wait()
            idx_v = idxs[rslot]
            for hg in range(2):
                slot = step % 2
                step += 1
                if out_pending[slot] is not None:
                    out_pending[slot].wait()
                buf = bufs[slot]
                hbase = hg * HG * NUM_REL

                @plsc.parallel_loop(0, VPR, 1, unroll=1)
                def _(v, buf=buf, idx_v=idx_v, hbase=hbase):
                    off = jnp.minimum(v * L, N - L)
                    for r in range(RG):
                        iv = idx_v[r, pl.ds(off, L)] + hbase
                        for h in range(HG):
                            vals = plsc.load_gather(
                                table_v, [iv + h * NUM_REL])
                            buf[h, r, pl.ds(off, L)] = vals

                out_pending[slot] = pltpu.async_copy(
                    buf,
                    out_hbm.at[pl.ds(hg * HG, HG),
                               pl.ds(row_base + rg * RG, RG), :],
                    osems[slot],
                )

        for p in out_pending:
            if p is not None:
                p.wait()

    return k(table, idx)


def kernel(relative_position_bias_table, relative_position_index):
    table = relative_position_bias_table.astype(jnp.float32).T.reshape(-1)
    idx = relative_position_index.astype(jnp.int32)
    idx = jnp.pad(idx, ((0, NPAD - N), (0, 0)))
    return _sc_bias_gather(table, idx)


# speedup vs baseline: 2.2791x; 1.0056x over previous
import functools

import jax
import jax.numpy as jnp
from jax import lax
from jax.experimental import pallas as pl
from jax.experimental.pallas import tpu as pltpu
from jax.experimental.pallas import tpu_sc as plsc

NUM_REL = 2212
H = 16
N = 577
NPAD = 584
NC = 2
NS = 16
L = 16

RG = 8
HG = 8
NRG = 3
VPR = 37


def _sc_bias_gather(table, idx):
    mesh = plsc.VectorSubcoreMesh(core_axis_name="c", subcore_axis_name="s")

    @functools.partial(
        pl.kernel,
        mesh=mesh,
        out_type=jax.ShapeDtypeStruct((H, N, N), jnp.float32),
        compiler_params=pltpu.CompilerParams(needs_layout_passes=False),
        scratch_types=[
            pltpu.VMEM((NUM_REL * H,), jnp.float32),
            pltpu.VMEM((RG, N), jnp.int32),
            pltpu.VMEM((RG, N), jnp.int32),
            pltpu.VMEM((HG, RG, N), jnp.float32),
            pltpu.VMEM((HG, RG, N), jnp.float32),
            pltpu.SemaphoreType.DMA,
            pltpu.SemaphoreType.DMA,
            pltpu.SemaphoreType.DMA,
            pltpu.SemaphoreType.DMA,
        ],
    )
    def k(table_hbm, idx_hbm, out_hbm, table_v,
          idx0, idx1, buf0, buf1, isem0, isem1, osem0, osem1):
        cid = lax.axis_index("c")
        sid = lax.axis_index("s")
        wid = sid * NC + cid
        row_base = ((wid * 70) // 31) * 8
        nxt = (((wid + 1) * 70) // 31) * 8
        want3 = jnp.logical_or(nxt - row_base >= 24, wid == NC * NS - 1)

        pltpu.sync_copy(table_hbm, table_v)

        idxs = (idx0, idx1)
        bufs = (buf0, buf1)
        isems = (isem0, isem1)
        osems = (osem0, osem1)
        out_pending = [None, None]
        idx_pending = [None, None]

        idx_pending[0] = pltpu.async_copy(
            idx_hbm.at[pl.ds(row_base, RG), :], idx0, isem0,
        )

        def do_group(rg, idx_v, step0, wait_prev):
            for hg in range(2):
                slot = (step0 + hg) % 2
                if wait_prev:
                    pltpu.make_async_copy(
                        bufs[slot],
                        out_hbm.at[pl.ds(hg * HG, HG), pl.ds(0, RG), :],
                        osems[slot],
                    ).wait()
                buf = bufs[slot]
                hbase = hg * HG * NUM_REL

                @plsc.parallel_loop(0, VPR, 1, unroll=1)
                def _(v, buf=buf, idx_v=idx_v, hbase=hbase):
                    off = jnp.minimum(v * L, N - L)
                    for r in range(RG):
                        iv = idx_v[r, pl.ds(off, L)] + hbase
                        for h in range(HG):
                            vals = plsc.load_gather(
                                table_v, [iv + h * NUM_REL])
                            buf[h, r, pl.ds(off, L)] = vals

                pltpu.async_copy(
                    buf,
                    out_hbm.at[pl.ds(hg * HG, HG),
                               pl.ds(row_base + rg * RG, RG), :],
                    osems[slot],
                )

        idx_pending[1] = pltpu.async_copy(
            idx_hbm.at[pl.ds(row_base + RG, RG), :], idx1, isem1,
        )
        idx_pending[0].wait()
        do_group(0, idx0, 0, False)
        idx_pending[0] = pltpu.async_copy(
            idx_hbm.at[pl.ds(row_base + 2 * RG, RG), :], idx0, isem0,
        )
        idx_pending[1].wait()
        do_group(1, idx1, 0, True)
        idx_pending[0].wait()
        @pl.when(want3)
        def _():
            do_group(2, idx0, 0, True)

        for slot in range(2):
            pltpu.make_async_copy(
                bufs[slot],
                out_hbm.at[pl.ds(slot * HG, HG), pl.ds(0, RG), :],
                osems[slot],
            ).wait()

    return k(table, idx)


def kernel(relative_position_bias_table, relative_position_index):
    table = relative_position_bias_table.astype(jnp.float32).T.reshape(-1)
    idx = relative_position_index.astype(jnp.int32)
    idx = jnp.pad(idx, ((0, NPAD - N), (0, 0)))
    return _sc_bias_gather(table, idx)


# speedup vs baseline: 2.3606x; 1.0358x over previous
import functools

import jax
import jax.numpy as jnp
from jax import lax
from jax.experimental import pallas as pl
from jax.experimental.pallas import tpu as pltpu
from jax.experimental.pallas import tpu_sc as plsc

NUM_REL = 2212
H = 16
N = 577
NPAD = 584
NC = 2
NS = 16
NW = NC * NS
L = 16

RG = 8
HG = 8
NTR = NPAD // RG
NU = NTR * 2
UMAX = 5
VPR = 37


def _sc_bias_gather(table, idx):
    mesh = plsc.VectorSubcoreMesh(core_axis_name="c", subcore_axis_name="s")

    @functools.partial(
        pl.kernel,
        mesh=mesh,
        out_type=jax.ShapeDtypeStruct((H, N, N), jnp.float32),
        compiler_params=pltpu.CompilerParams(needs_layout_passes=False),
        scratch_types=[
            pltpu.VMEM((NUM_REL * H,), jnp.float32),
            pltpu.VMEM((RG, N), jnp.int32),
            pltpu.VMEM((RG, N), jnp.int32),
            pltpu.VMEM((HG, RG, N), jnp.float32),
            pltpu.VMEM((HG, RG, N), jnp.float32),
            pltpu.SemaphoreType.DMA,
            pltpu.SemaphoreType.DMA,
            pltpu.SemaphoreType.DMA,
            pltpu.SemaphoreType.DMA,
        ],
    )
    def k(table_hbm, idx_hbm, out_hbm, table_v,
          idx0, idx1, buf0, buf1, isem0, isem1, osem0, osem1):
        cid = lax.axis_index("c")
        sid = lax.axis_index("s")
        wid = sid * NC + cid
        u_start = (wid * NU) // NW
        n_u = ((wid + 1) * NU) // NW - u_start

        pltpu.sync_copy(table_hbm, table_v)

        idxs = (idx0, idx1)
        bufs = (buf0, buf1)
        isems = (isem0, isem1)
        osems = (osem0, osem1)

        def stage(k_, u):
            pltpu.async_copy(
                idx_hbm.at[pl.ds((u // 2) * RG, RG), :],
                idxs[k_ % 2], isems[k_ % 2],
            )

        stage(0, u_start)

        for k_ in range(UMAX):
            @pl.when(k_ < n_u)
            def _(k_=k_):
                u = u_start + k_
                tr = u // 2
                hg = u % 2
                slot = k_ % 2
                if k_ + 1 < UMAX:
                    @pl.when(k_ + 1 < n_u)
                    def _():
                        stage(k_ + 1, u + 1)
                pltpu.make_async_copy(
                    idx_hbm.at[pl.ds(0, RG), :], idxs[slot], isems[slot]
                ).wait()
                if k_ >= 2:
                    pltpu.make_async_copy(
                        bufs[slot], out_hbm.at[pl.ds(0, HG), pl.ds(0, RG), :],
                        osems[slot],
                    ).wait()
                idx_v = idxs[slot]
                buf = bufs[slot]
                hbase = hg * (HG * NUM_REL)

                @plsc.parallel_loop(0, VPR, 1, unroll=1)
                def _(v, buf=buf, idx_v=idx_v, hbase=hbase):
                    off = jnp.minimum(v * L, N - L)
                    for r in range(RG):
                        iv = idx_v[r, pl.ds(off, L)] + hbase
                        for h in range(HG):
                            vals = plsc.load_gather(
                                table_v, [iv + h * NUM_REL])
                            buf[h, r, pl.ds(off, L)] = vals

                pltpu.async_copy(
                    buf,
                    out_hbm.at[pl.ds(hg * HG, HG), pl.ds(tr * RG, RG), :],
                    osems[slot],
                )

        for slot in range(2):
            pltpu.make_async_copy(
                bufs[slot], out_hbm.at[pl.ds(0, HG), pl.ds(0, RG), :],
                osems[slot],
            ).wait()

    return k(table, idx)


def kernel(relative_position_bias_table, relative_position_index):
    table = relative_position_bias_table.astype(jnp.float32).T.reshape(-1)
    idx = relative_position_index.astype(jnp.int32)
    idx = jnp.pad(idx, ((0, NPAD - N), (0, 0)))
    return _sc_bias_gather(table, idx)
